# baseline (speedup 1.0000x reference)
import functools

import jax
import jax.numpy as jnp
from jax import lax
from jax.experimental import pallas as pl
from jax.experimental.pallas import tpu as pltpu
from jax.experimental.pallas import tpu_sc as plsc

D_MODEL = 512
N_ROWS = 4096
N_COLS = 50

_info = plsc.get_sparse_core_info()
_NC, _NS = _info.num_cores, _info.num_subcores
_NW = _NC * _NS
_IBLK = N_ROWS // _NW
_CHUNK = 32
_NBUF = 6
_CPP = _IBLK // _CHUNK
_N_ITEMS = N_COLS * _CPP


def _build():
    mesh = plsc.VectorSubcoreMesh(core_axis_name="c", subcore_axis_name="s")

    @functools.partial(
        pl.kernel,
        out_type=jax.ShapeDtypeStruct((N_COLS, N_ROWS, D_MODEL), jnp.float32),
        mesh=mesh,
        scratch_types=[
            pltpu.VMEM((N_COLS, _IBLK), jnp.int32),
            pltpu.SemaphoreType.DMA,
        ]
        + [pltpu.VMEM((_CHUNK, D_MODEL), jnp.float32)] * _NBUF
        + [pltpu.SemaphoreType.DMA] * (2 * _NBUF),
    )
    def emb(idx_hbm, table_hbm, out_hbm, idx_v, isem, *bufs_and_sems):
        bufs = bufs_and_sems[:_NBUF]
        gsems = bufs_and_sems[_NBUF:2 * _NBUF]
        wsems = bufs_and_sems[2 * _NBUF:]
        wid = lax.axis_index("s") * _NC + lax.axis_index("c")
        ibase = wid * _IBLK

        pltpu.make_async_copy(
            idx_hbm.at[:, pl.ds(ibase, _IBLK)], idx_v, isem).start()
        pltpu.make_async_copy(
            idx_hbm.at[:, pl.ds(0, _IBLK)], idx_v, isem).wait()

        def g_start(t, b):
            pltpu.make_async_copy(
                table_hbm.at[idx_v.at[t // _CPP, pl.ds((t % _CPP) * _CHUNK, _CHUNK)]],
                bufs[b], gsems[b]).start()

        def g_wait(b):
            pltpu.make_async_copy(
                table_hbm.at[idx_v.at[0, pl.ds(0, _CHUNK)]],
                bufs[b], gsems[b]).wait()

        def w_start(t, b):
            j = t // _CPP
            c = t % _CPP
            pltpu.make_async_copy(
                bufs[b], out_hbm.at[j, pl.ds(ibase + c * _CHUNK, _CHUNK)],
                wsems[b]).start()

        def w_wait(b):
            pltpu.make_async_copy(
                bufs[b], out_hbm.at[0, pl.ds(0, _CHUNK)], wsems[b]).wait()

        n_steps = _N_ITEMS // _NBUF
        ahead = 4
        drain_lag = _NBUF - ahead
        for k in range(ahead):
            g_start(k, k)

        def body(i, carry):
            t0 = i * _NBUF
            for b in range(_NBUF):
                t = t0 + b
                g_wait(b)
                w_start(t, b)
                bn = (b + ahead) % _NBUF

                @pl.when(t + ahead < _N_ITEMS)
                def _arm():
                    @pl.when(t >= drain_lag)
                    def _drain():
                        w_wait(bn)

                    g_start(t + ahead, bn)

            return carry

        lax.fori_loop(0, n_steps, body, 0)
        for t in range(n_steps * _NBUF, _N_ITEMS):
            g_wait(t % _NBUF)
            w_start(t, t % _NBUF)
        for b in range(_NBUF):
            w_wait(b)

    return emb


_emb = _build()


def kernel(x, W):
    xt = jnp.transpose(x.astype(jnp.int32))
    return jnp.transpose(_emb(xt, W), (1, 0, 2))
